# Initial kernel scaffold; baseline (speedup 1.0000x reference)
import jax
import jax.numpy as jnp
from jax.experimental import pallas as pl


def kernel(x, router_w, router_b, w_gate, w_up, w_down):
    raise NotImplementedError("write your pallas kernel here")


# speedup vs baseline: 1.3427x; 1.3427x over previous
import functools

import jax
import jax.numpy as jnp
from jax import lax
from jax.experimental import pallas as pl
from jax.experimental.pallas import tpu as pltpu
from jax.experimental.pallas import tpu_sc as plsc

_T, _D, _E, _F, _K = 2048, 1024, 8, 2048, 2
_M = 256
_NT = (_T * _K) // _M + _E - 1
_NP = _NT * _M
_FT = 512
_NF = _F // _FT
_NTPAD = 32

_NC, _NS = 2, 16
_NW = _NC * _NS
_TPW = _T // _NW
_CW = 32


def _router_body(x_ref, rw_ref, rb_ref, pos_ref, aff_ref, meta_ref,
                 oh_ref, rank_ref):
    xv = x_ref[...]
    logits = jnp.dot(xv, rw_ref[...], preferred_element_type=jnp.float32)
    logits = logits + rb_ref[...]
    iota_e = lax.broadcasted_iota(jnp.int32, (_T, _E), 1)
    m1 = jnp.max(logits, axis=1, keepdims=True)
    i1 = jnp.min(jnp.where(logits == m1, iota_e, _E), axis=1, keepdims=True)
    masked = jnp.where(iota_e == i1, -jnp.inf, logits)
    m2 = jnp.max(masked, axis=1, keepdims=True)
    i2 = jnp.min(jnp.where(masked == m2, iota_e, _E), axis=1, keepdims=True)
    ex = jnp.exp(m2 - m1)
    a0 = 1.0 / (1.0 + ex)
    a1 = ex * a0
    oh0 = (iota_e == i1).astype(jnp.float32)
    oh1 = (iota_e == i2).astype(jnp.float32)
    oh_ref[0:_T, :] = oh0
    oh_ref[_T:2 * _T, :] = oh1
    bl = 512
    tri = (lax.broadcasted_iota(jnp.int32, (bl, bl), 1)
           < lax.broadcasted_iota(jnp.int32, (bl, bl), 0)).astype(jnp.float32)
    carry = jnp.zeros((1, _E), jnp.float32)
    for b in range(2 * _T // bl):
        blk = oh_ref[b * bl:(b + 1) * bl, :]
        rank_ref[b * bl:(b + 1) * bl, :] = (
            jnp.dot(tri, blk, preferred_element_type=jnp.float32) + carry)
        carry = carry + jnp.sum(blk, axis=0, keepdims=True)
    counts = carry
    tiles_e = jnp.ceil(counts / _M)
    tri_e = (lax.broadcasted_iota(jnp.int32, (_E, _E), 0)
             < lax.broadcasted_iota(jnp.int32, (_E, _E), 1)).astype(jnp.float32)
    start_tile = jnp.dot(tiles_e, tri_e, preferred_element_type=jnp.float32)
    off_pad = start_tile * _M
    rk0 = jnp.sum(rank_ref[0:_T, :] * oh0, axis=1, keepdims=True)
    rk1 = jnp.sum(rank_ref[_T:2 * _T, :] * oh1, axis=1, keepdims=True)
    base0 = jnp.sum(off_pad * oh0, axis=1, keepdims=True)
    base1 = jnp.sum(off_pad * oh1, axis=1, keepdims=True)
    pos0 = (base0 + rk0).astype(jnp.int32)
    pos1 = (base1 + rk1).astype(jnp.int32)
    pos_ref[...] = jnp.concatenate([pos0, pos1], axis=1)
    aff_ref[...] = jnp.concatenate([a0, a1], axis=1)
    end_tile = start_tile + tiles_e
    iota_t = lax.broadcasted_iota(jnp.int32, (1, _NTPAD), 1).astype(jnp.float32)
    te = jnp.zeros((1, _NTPAD), jnp.float32)
    for e in range(_E):
        te = te + (iota_t >= end_tile[:, e:e + 1]).astype(jnp.float32)
    te = jnp.minimum(te, float(_E - 1))
    valid = (iota_t < end_tile[:, _E - 1:_E]).astype(jnp.float32)
    rows = [te, valid] + [jnp.zeros((1, _NTPAD), jnp.float32)] * 6
    meta_ref[...] = jnp.concatenate(rows, axis=0).astype(jnp.int32)


@functools.cache
def _router_kernel():
    return pl.pallas_call(
        _router_body,
        out_shape=[
            jax.ShapeDtypeStruct((_T, _K), jnp.int32),
            jax.ShapeDtypeStruct((_T, _K), jnp.float32),
            jax.ShapeDtypeStruct((8, _NTPAD), jnp.int32),
        ],
        scratch_shapes=[
            pltpu.VMEM((2 * _T, _E), jnp.float32),
            pltpu.VMEM((2 * _T, _E), jnp.float32),
        ],
    )


def _dispatch_body(x_hbm, p0_hbm, p1_hbm, xs_hbm, idx0_v, idx1_v, rows_v, sem):
    wid = lax.axis_index("s") * _NC + lax.axis_index("c")
    base = wid * _TPW
    pltpu.sync_copy(p0_hbm.at[pl.ds(base, _TPW)], idx0_v)
    pltpu.sync_copy(p1_hbm.at[pl.ds(base, _TPW)], idx1_v)
    pltpu.sync_copy(x_hbm.at[pl.ds(base, _TPW)], rows_v)
    pltpu.async_copy(rows_v, xs_hbm.at[idx0_v], sem).wait()
    pltpu.async_copy(rows_v, xs_hbm.at[idx1_v], sem).wait()


@functools.cache
def _dispatch_kernel():
    return pl.kernel(
        _dispatch_body,
        out_type=jax.ShapeDtypeStruct((_NP, _D), jnp.float32),
        mesh=plsc.VectorSubcoreMesh(core_axis_name="c", subcore_axis_name="s",
                                    num_cores=_NC, num_subcores=_NS),
        scratch_types=[
            pltpu.VMEM((_TPW,), jnp.int32),
            pltpu.VMEM((_TPW,), jnp.int32),
            pltpu.VMEM((_TPW, _D), jnp.float32),
            pltpu.SemaphoreType.DMA,
        ],
    )


def _dispatch_call(t, pos0, pos1):
    return _dispatch_kernel()(t, pos0, pos1)


def _mlp_body(meta_ref, x_ref, wg_ref, wu_ref, wd_ref, y_ref, acc_ref):
    i = pl.program_id(0)
    f = pl.program_id(1)
    valid = meta_ref[1, i] == 1

    @pl.when(valid)
    def _():
        xv = x_ref[...]
        g = jnp.dot(xv, wg_ref[0], preferred_element_type=jnp.float32)
        u = jnp.dot(xv, wu_ref[0], preferred_element_type=jnp.float32)
        h = g * (1.0 / (1.0 + jnp.exp(-g))) * u
        part = jnp.dot(h, wd_ref[0], preferred_element_type=jnp.float32)

        @pl.when(f == 0)
        def _():
            acc_ref[...] = part

        @pl.when(f > 0)
        def _():
            acc_ref[...] += part

        @pl.when(f == _NF - 1)
        def _():
            y_ref[...] = acc_ref[...]


@functools.cache
def _mlp_kernel():
    return pl.pallas_call(
        _mlp_body,
        grid_spec=pltpu.PrefetchScalarGridSpec(
            num_scalar_prefetch=1,
            grid=(_NT, _NF),
            in_specs=[
                pl.BlockSpec((_M, _D), lambda i, f, m: (i, 0)),
                pl.BlockSpec((1, _D, _FT), lambda i, f, m: (m[0, i], 0, f)),
                pl.BlockSpec((1, _D, _FT), lambda i, f, m: (m[0, i], 0, f)),
                pl.BlockSpec((1, _FT, _D), lambda i, f, m: (m[0, i], f, 0)),
            ],
            out_specs=pl.BlockSpec((_M, _D), lambda i, f, m: (i, 0)),
            scratch_shapes=[pltpu.VMEM((_M, _D), jnp.float32)],
        ),
        out_shape=jax.ShapeDtypeStruct((_NP, _D), jnp.float32),
        compiler_params=pltpu.CompilerParams(
            dimension_semantics=("arbitrary", "arbitrary")),
    )


def _combine_body(y_hbm, p0_hbm, p1_hbm, a0_hbm, a1_hbm, out_hbm,
                  idx0_v, idx1_v, aff0_v, aff1_v, buf0_v, buf1_v, sem):
    wid = lax.axis_index("s") * _NC + lax.axis_index("c")
    for c in range(_TPW // _CW):
        base = wid * _TPW + c * _CW
        pltpu.sync_copy(p0_hbm.at[pl.ds(base, _CW)], idx0_v)
        pltpu.sync_copy(p1_hbm.at[pl.ds(base, _CW)], idx1_v)
        pltpu.sync_copy(a0_hbm.at[pl.ds(base, _CW)], aff0_v)
        pltpu.sync_copy(a1_hbm.at[pl.ds(base, _CW)], aff1_v)
        pltpu.async_copy(y_hbm.at[idx0_v], buf0_v, sem).wait()
        pltpu.async_copy(y_hbm.at[idx1_v], buf1_v, sem).wait()

        for c16 in range(_CW // 16):
            av0 = aff0_v[pl.ds(c16 * 16, 16)]
            av1 = aff1_v[pl.ds(c16 * 16, 16)]

            def row_body(rr, carry, av0=av0, av1=av1, c16=c16):
                z16 = jnp.zeros((16,), jnp.int32)
                a0 = av0.at[z16 + rr].get(mode="promise_in_bounds")
                a1 = av1.at[z16 + rr].get(mode="promise_in_bounds")
                r = c16 * 16 + rr
                for j in range(_D // 16):
                    sl = pl.ds(j * 16, 16)
                    buf0_v[r, sl] = a0 * buf0_v[r, sl] + a1 * buf1_v[r, sl]
                return carry

            lax.fori_loop(0, 16, row_body, 0)
        pltpu.sync_copy(buf0_v, out_hbm.at[pl.ds(base, _CW)])


@functools.cache
def _combine_kernel():
    return pl.kernel(
        _combine_body,
        out_type=jax.ShapeDtypeStruct((_T, _D), jnp.float32),
        mesh=plsc.VectorSubcoreMesh(core_axis_name="c", subcore_axis_name="s",
                                    num_cores=_NC, num_subcores=_NS),
        scratch_types=[
            pltpu.VMEM((_CW,), jnp.int32),
            pltpu.VMEM((_CW,), jnp.int32),
            pltpu.VMEM((_CW,), jnp.float32),
            pltpu.VMEM((_CW,), jnp.float32),
            pltpu.VMEM((_CW, _D), jnp.float32),
            pltpu.VMEM((_CW, _D), jnp.float32),
            pltpu.SemaphoreType.DMA,
        ],
    )


def _combine_call(y_sorted, pos0, pos1, a0, a1):
    return _combine_kernel()(y_sorted, pos0, pos1, a0, a1)


def kernel(x, router_w, router_b, w_gate, w_up, w_down):
    t = x.reshape(_T, _D)
    pos, aff, meta = _router_kernel()(t, router_w, router_b.reshape(1, _E))
    pos0, pos1 = pos[:, 0], pos[:, 1]
    x_sorted = _dispatch_call(t, pos0, pos1)
    y_sorted = _mlp_kernel()(meta, x_sorted, w_gate, w_up, w_down)
    out = _combine_call(y_sorted, pos0, pos1, aff[:, 0], aff[:, 1])
    return out.reshape(x.shape)
